# baseline (speedup 1.0000x reference)
import jax
import jax.numpy as jnp
from jax.experimental import pallas as pl
from jax.experimental.pallas import tpu as pltpu

_A = 1000
_B = 8
_C = 125
_SEG = 125
_NSEG = _A // _SEG
_DEPTH = 4


def _tc_strided_body(ent_in, rel_in, ent_out, rel_out, bufs, relbuf, sem_in, sem_out):
    ent3_in = ent_in.reshape(_A, _B, _C, 64)
    ent3_out = ent_out.reshape(_A, _B, _C, 64)

    pieces = [(s, p) for p in range(_B) for s in range(_NSEG)]

    def src(s, p):
        return ent3_in.at[pl.ds(s * _SEG, _SEG), p]

    def dst(s, p):
        return ent3_out.at[pl.ds(s * _SEG, _SEG), p]

    n = len(pieces)
    in_h = [None] * _DEPTH
    out_h = [None] * _DEPTH
    for b in range(_DEPTH):
        s, p = pieces[b]
        in_h[b] = pltpu.make_async_copy(src(s, p), bufs.at[b], sem_in)
        in_h[b].start()
    for j in range(n):
        b = j % _DEPTH
        s, p = pieces[j]
        in_h[b].wait()
        out_h[b] = pltpu.make_async_copy(bufs.at[b], dst(s, p), sem_out)
        out_h[b].start()
        nxt = j + _DEPTH
        if nxt < n:
            out_h[b].wait()
            s2, p2 = pieces[nxt]
            in_h[b] = pltpu.make_async_copy(src(s2, p2), bufs.at[b], sem_in)
            in_h[b].start()
    for j in range(max(0, n - _DEPTH), n):
        out_h[j % _DEPTH].wait()

    rel_in_h = pltpu.make_async_copy(rel_in.at[...], relbuf, sem_in)
    rel_in_h.start()
    rel_in_h.wait()
    rel_out_h = pltpu.make_async_copy(relbuf, rel_out.at[...], sem_out)
    rel_out_h.start()
    rel_out_h.wait()


def kernel(x_dict, edge_index, entity_emb, rel_emb):
    ent_out, rel_out = pl.pallas_call(
        _tc_strided_body,
        out_shape=(
            jax.ShapeDtypeStruct(entity_emb.shape, entity_emb.dtype),
            jax.ShapeDtypeStruct(rel_emb.shape, rel_emb.dtype),
        ),
        in_specs=[
            pl.BlockSpec(memory_space=pl.ANY),
            pl.BlockSpec(memory_space=pl.ANY),
        ],
        out_specs=(
            pl.BlockSpec(memory_space=pl.ANY),
            pl.BlockSpec(memory_space=pl.ANY),
        ),
        scratch_shapes=[
            pltpu.VMEM((_DEPTH, _SEG, _C, 64), jnp.float32),
            pltpu.VMEM((1000, 64), jnp.float32),
            pltpu.SemaphoreType.DMA,
            pltpu.SemaphoreType.DMA,
        ],
    )(entity_emb, rel_emb)
    return (ent_out, rel_out)
